# baseline (speedup 1.0000x reference)
import math

import jax
import jax.numpy as jnp
from jax import lax
from jax.experimental import pallas as pl
from jax.experimental.pallas import tpu as pltpu
from jax.experimental.pallas import tpu_sc as plsc

VOCAB = 1000000
D = 128
BATCH = 4096
SEQ = 200

NC = 2
NS = 16
NW = NC * NS
B = BATCH * SEQ
B_PER_W = B // NW
CHUNK = 128
NCHUNK = B_PER_W // CHUNK
NB = 5
GAHEAD = 3
LANES = 16
SCALE = math.sqrt(D)


def _emb_body(table_hbm, idx_hbm, out_hbm, idx_v, bufs, gsems, ssems):
    wid = lax.axis_index("s") * NC + lax.axis_index("c")
    base = wid * B_PER_W

    pltpu.sync_copy(idx_hbm.at[wid], idx_v)

    for g in range(GAHEAD):
        pltpu.async_copy(table_hbm.at[idx_v.at[g]], bufs[g], gsems[g])

    def scale_chunk(buf):
        @plsc.parallel_loop(0, CHUNK, step=1, unroll=8)
        def _row(r):
            for c in range(D // LANES):
                sl = pl.ds(c * LANES, LANES)
                buf[r, sl] = buf[r, sl] * SCALE

    def step(it, _):
        j0 = NB * it
        for k in range(NB):
            j = j0 + k
            buf, gsem, ssem = bufs[k], gsems[k], ssems[k]
            kw = (k - 2) % NB
            pltpu.make_async_copy(table_hbm.at[idx_v.at[j]], buf, gsem).wait()
            scale_chunk(buf)
            pltpu.async_copy(
                buf, out_hbm.at[pl.ds(base + j * CHUNK, CHUNK)], ssem)

            @pl.when(j >= 2)
            def _():
                pltpu.make_async_copy(
                    bufs[kw],
                    out_hbm.at[pl.ds(base + (j - 2) * CHUNK, CHUNK)],
                    ssems[kw]).wait()

            @pl.when(j < NCHUNK - GAHEAD)
            def _():
                pltpu.async_copy(
                    table_hbm.at[idx_v.at[j + GAHEAD]], bufs[kw], gsems[kw])
        return 0

    lax.fori_loop(0, NCHUNK // NB, step, 0)

    for j in (NCHUNK - 2, NCHUNK - 1):
        pltpu.make_async_copy(
            bufs[j % NB], out_hbm.at[pl.ds(base + j * CHUNK, CHUNK)],
            ssems[j % NB]).wait()


@jax.jit
def kernel(x, table):
    mesh = plsc.VectorSubcoreMesh(core_axis_name="c", subcore_axis_name="s")
    fn = pl.kernel(
        _emb_body,
        out_type=jax.ShapeDtypeStruct((B, D), jnp.float32),
        mesh=mesh,
        scratch_types=[
            pltpu.VMEM((NCHUNK, CHUNK), jnp.int32),
            [pltpu.VMEM((CHUNK, D), jnp.float32) for _ in range(NB)],
            [pltpu.SemaphoreType.DMA for _ in range(NB)],
            [pltpu.SemaphoreType.DMA for _ in range(NB)],
        ],
        name="sc_embedding_lookup",
    )
    idx = x.reshape(NW, NCHUNK, CHUNK)
    out = fn(table, idx)
    return out.reshape(BATCH, SEQ, D)
